# baseline (speedup 1.0000x reference)
import jax
import jax.numpy as jnp
from jax.experimental import pallas as pl

_NUM_TOKENS = 100
_D_MODEL = 4096
_BATCH = 4


def _body(table_ref, out_ref):
    for b in range(_BATCH):
        out_ref[b] = table_ref[...]


def kernel(batch_size, prompt_embeddings):
    del batch_size
    return pl.pallas_call(
        _body,
        out_shape=jax.ShapeDtypeStruct(
            (_BATCH, _NUM_TOKENS, _D_MODEL), jnp.float32
        ),
    )(prompt_embeddings)
